# baseline (speedup 1.0000x reference)
import functools

import jax
import jax.numpy as jnp
from jax import lax
from jax.experimental import pallas as pl
from jax.experimental.pallas import tpu as pltpu
from jax.experimental.pallas import tpu_sc as plsc

N = 10000
E = 160000
G = 100
BN = 1.0 / (1.0 + 1e-5) ** 0.5

NC = 2
NS = 16
NW = NC * NS
CK = 128
NCHUNK = E // CK
ITERS = (NCHUNK + NW - 1) // NW


def _leaky(x):
    return jnp.where(x >= 0, x, 0.01 * x)


def _pad128(w):
    return jnp.pad(w, ((0, 0), (0, 128 - w.shape[1])))


BE = 256


def _curve_body(x_ref, w1_ref, w2_ref, w3_ref, fc_ref, pw_ref, pb_ref,
                crv_ref, ep_ref, s0, s1, s2, s3):
    b = BE

    def fill(s, val, c):
        s[pl.ds(0, b), :] = jnp.zeros((b, c), jnp.float32)
        s[pl.ds(11 * b, b), :] = jnp.zeros((b, c), jnp.float32)
        s[pl.ds(b, 10 * b), :] = val

    fill(s0, x_ref[...].reshape(10 * b, 3), 3)

    def layer(s_in, s_out, w_ref, cout):
        acc = jnp.dot(s_in[pl.ds(0, 10 * b), :], w_ref[0],
                      preferred_element_type=jnp.float32)
        acc = acc + jnp.dot(s_in[pl.ds(b, 10 * b), :], w_ref[1],
                            preferred_element_type=jnp.float32)
        acc = acc + jnp.dot(s_in[pl.ds(2 * b, 10 * b), :], w_ref[2],
                            preferred_element_type=jnp.float32)
        fill(s_out, _leaky(acc * BN), cout)

    layer(s0, s1, w1_ref, 64)
    layer(s1, s2, w2_ref, 128)
    layer(s2, s3, w3_ref, 256)

    m = s3[pl.ds(b, b), :]
    for l in range(1, 10):
        m = m + s3[pl.ds((1 + l) * b, b), :]
    m = m * 0.1
    crv = _leaky(jnp.dot(m, fc_ref[...], preferred_element_type=jnp.float32) * BN)
    crv_ref[...] = crv
    ep_ref[...] = jnp.dot(crv, pw_ref[...],
                          preferred_element_type=jnp.float32) + pb_ref[...]


def _run_curve(efeat, cw1, cw2, cw3, cfc, pw0, pb0):
    xt = jnp.transpose(efeat, (2, 0, 1))
    grid = E // BE
    return pl.pallas_call(
        _curve_body,
        grid=(grid,),
        in_specs=[
            pl.BlockSpec((10, BE, 3), lambda i: (0, i, 0)),
            pl.BlockSpec((3, 3, 64), lambda i: (0, 0, 0)),
            pl.BlockSpec((3, 64, 128), lambda i: (0, 0, 0)),
            pl.BlockSpec((3, 128, 256), lambda i: (0, 0, 0)),
            pl.BlockSpec((256, 64), lambda i: (0, 0)),
            pl.BlockSpec((64, 128), lambda i: (0, 0)),
            pl.BlockSpec((1, 128), lambda i: (0, 0)),
        ],
        out_specs=[
            pl.BlockSpec((BE, 64), lambda i: (i, 0)),
            pl.BlockSpec((BE, 128), lambda i: (i, 0)),
        ],
        out_shape=[
            jax.ShapeDtypeStruct((E, 64), jnp.float32),
            jax.ShapeDtypeStruct((E, 128), jnp.float32),
        ],
        scratch_shapes=[
            pltpu.VMEM((12 * BE, 3), jnp.float32),
            pltpu.VMEM((12 * BE, 64), jnp.float32),
            pltpu.VMEM((12 * BE, 128), jnp.float32),
            pltpu.VMEM((12 * BE, 256), jnp.float32),
        ],
    )(xt, jnp.transpose(cw1, (2, 1, 0)), jnp.transpose(cw2, (2, 1, 0)),
      jnp.transpose(cw3, (2, 1, 0)), cfc.T, _pad128(pw0.T),
      _pad128(pb0.reshape(1, 64)))


BS = 16


def _surface_body(x_ref, w1_ref, w2_ref, w3_ref, fc_ref, srf_ref, s0, s1, s2, s3):
    b = BS
    rows = 118 * b

    x2 = x_ref[...].reshape(100 * b, 4)
    s0[...] = jnp.zeros((144 * b, 4), jnp.float32)
    for h in range(10):
        s0[pl.ds(((h + 1) * 12 + 1) * b, 10 * b), :] = x2[h * 10 * b:(h + 1) * 10 * b, :]

    def layer(s_in, s_out, w_ref, cout):
        acc = None
        for dy in range(3):
            for dx in range(3):
                sh = ((dy - 1) * 12 + (dx - 1)) * b
                t = jnp.dot(s_in[pl.ds(13 * b + sh, rows), :], w_ref[dy, dx],
                            preferred_element_type=jnp.float32)
                acc = t if acc is None else acc + t
        y = _leaky(acc * BN)
        ridx = lax.broadcasted_iota(jnp.int32, (rows, cout), 0)
        pos = ridx // b + 13
        hh = pos // 12
        ww = pos - hh * 12
        valid = (hh >= 1) & (hh <= 10) & (ww >= 1) & (ww <= 10)
        y = jnp.where(valid, y, 0.0)
        s_out[pl.ds(0, 13 * b), :] = jnp.zeros((13 * b, cout), jnp.float32)
        s_out[pl.ds(131 * b, 13 * b), :] = jnp.zeros((13 * b, cout), jnp.float32)
        s_out[pl.ds(13 * b, rows), :] = y

    layer(s0, s1, w1_ref, 64)
    layer(s1, s2, w2_ref, 128)
    layer(s2, s3, w3_ref, 256)

    v = s3[...]
    a = v[0:12 * b, :]
    for p in range(1, 12):
        a = a + v[p * 12 * b:(p + 1) * 12 * b, :]
    mrow = a[0:b, :]
    for q in range(1, 12):
        mrow = mrow + a[q * b:(q + 1) * b, :]
    m = mrow * 0.01
    srf_ref[...] = _leaky(jnp.dot(m, fc_ref[...],
                                  preferred_element_type=jnp.float32) * BN)


def _run_surface(nfeat, sw1, sw2, sw3, sfc):
    xt = jnp.transpose(nfeat, (2, 3, 0, 1))
    grid = N // BS
    return pl.pallas_call(
        _surface_body,
        grid=(grid,),
        in_specs=[
            pl.BlockSpec((10, 10, BS, 4), lambda i: (0, 0, i, 0)),
            pl.BlockSpec((3, 3, 4, 64), lambda i: (0, 0, 0, 0)),
            pl.BlockSpec((3, 3, 64, 128), lambda i: (0, 0, 0, 0)),
            pl.BlockSpec((3, 3, 128, 256), lambda i: (0, 0, 0, 0)),
            pl.BlockSpec((256, 128), lambda i: (0, 0)),
        ],
        out_specs=pl.BlockSpec((BS, 128), lambda i: (i, 0)),
        out_shape=jax.ShapeDtypeStruct((N, 128), jnp.float32),
        scratch_shapes=[
            pltpu.VMEM((144 * BS, 4), jnp.float32),
            pltpu.VMEM((144 * BS, 64), jnp.float32),
            pltpu.VMEM((144 * BS, 128), jnp.float32),
            pltpu.VMEM((144 * BS, 256), jnp.float32),
        ],
    )(xt, jnp.transpose(sw1, (2, 3, 1, 0)), jnp.transpose(sw2, (2, 3, 1, 0)),
      jnp.transpose(sw3, (2, 3, 1, 0)), _pad128(sfc.T))


def _seg_body(h_hbm, ep_hbm, src_hbm, dst_hbm, zeros_hbm, out_hbm,
              src_v, dst_v, rows_v, ep_v, shared):
    cidx = lax.axis_index("c")
    sid = lax.axis_index("s")
    w = sid * NC + cidx

    @pl.when(sid < 10)
    def _():
        pltpu.sync_copy(zeros_hbm, shared.at[pl.ds(sid * 1000, 1000)])

    plsc.subcore_barrier()

    def body(it, _):
        cid = it * NW + w

        @pl.when(cid < NCHUNK)
        def _():
            base = cid * CK
            pltpu.sync_copy(src_hbm.at[pl.ds(base, CK)], src_v)
            pltpu.sync_copy(dst_hbm.at[pl.ds(base, CK)], dst_v)
            pltpu.sync_copy(h_hbm.at[src_v], rows_v)
            pltpu.sync_copy(ep_hbm.at[pl.ds(base, CK)], ep_v)
            pltpu.sync_copy(rows_v, shared.at[dst_v], add=True)
            pltpu.sync_copy(ep_v, shared.at[dst_v], add=True)
        return 0

    lax.fori_loop(0, ITERS, body, 0)
    plsc.subcore_barrier()

    @pl.when(sid < 10)
    def _():
        pltpu.sync_copy(shared.at[pl.ds(sid * 1000, 1000)],
                        out_hbm.at[pl.ds(cidx * N + sid * 1000, 1000)])


def _run_seg(h, ep, src, dst):
    zeros = jnp.zeros((1000, 128), jnp.float32)
    mesh = plsc.VectorSubcoreMesh(core_axis_name="c", subcore_axis_name="s")
    fn = functools.partial(
        pl.kernel, mesh=mesh,
        out_type=jax.ShapeDtypeStruct((2 * N, 128), jnp.float32),
        scratch_types=[
            pltpu.VMEM((CK,), jnp.int32),
            pltpu.VMEM((CK,), jnp.int32),
            pltpu.VMEM((CK, 128), jnp.float32),
            pltpu.VMEM((CK, 128), jnp.float32),
            pltpu.VMEM_SHARED((N, 128), jnp.float32),
        ],
    )(_seg_body)
    return fn(h, ep, src, dst, zeros)


def _g2_body(t_hbm, src_hbm, dst_hbm, g1_hbm, g2_hbm, src_v, dst_v, r1_v, r2_v):
    cidx = lax.axis_index("c")
    sid = lax.axis_index("s")
    w = sid * NC + cidx

    def body(it, _):
        cid = it * NW + w

        @pl.when(cid < NCHUNK)
        def _():
            base = cid * CK
            pltpu.sync_copy(src_hbm.at[pl.ds(base, CK)], src_v)
            pltpu.sync_copy(dst_hbm.at[pl.ds(base, CK)], dst_v)
            pltpu.sync_copy(t_hbm.at[src_v], r1_v)
            pltpu.sync_copy(t_hbm.at[dst_v], r2_v)
            pltpu.sync_copy(r1_v, g1_hbm.at[pl.ds(base, CK)])
            pltpu.sync_copy(r2_v, g2_hbm.at[pl.ds(base, CK)])
        return 0

    lax.fori_loop(0, ITERS, body, 0)


def _run_gather2(table, src, dst):
    mesh = plsc.VectorSubcoreMesh(core_axis_name="c", subcore_axis_name="s")
    fn = functools.partial(
        pl.kernel, mesh=mesh,
        out_type=[jax.ShapeDtypeStruct((E, 128), jnp.float32),
                  jax.ShapeDtypeStruct((E, 128), jnp.float32)],
        scratch_types=[
            pltpu.VMEM((CK,), jnp.int32),
            pltpu.VMEM((CK,), jnp.int32),
            pltpu.VMEM((CK, 128), jnp.float32),
            pltpu.VMEM((CK, 128), jnp.float32),
        ],
    )(_g2_body)
    return fn(table, src, dst)


BR = 2000


def _node_body(h_ref, a0_ref, a1_ref, eps_ref, w1_ref, b1_ref, w2_ref, b2_ref,
               pw_ref, hn_ref, hnp_ref):
    x = (eps_ref[0, 0] * h_ref[...][:, 0:64] + a0_ref[...][:, 0:64]
         + a1_ref[...][:, 0:64])
    hh = jnp.maximum((jnp.dot(x, w1_ref[...], preferred_element_type=jnp.float32)
                      + b1_ref[...]) * BN, 0.0)
    y = _leaky((jnp.dot(hh, w2_ref[...], preferred_element_type=jnp.float32)
                + b2_ref[...]) * BN)
    hn_ref[...] = y
    hnp_ref[...] = jnp.dot(y[:, 0:64], pw_ref[...],
                           preferred_element_type=jnp.float32)


def _run_node(h, a0, a1, nc, ec_pw):
    grid = N // BR
    full = lambda shape: pl.BlockSpec(shape, lambda i: tuple(0 for _ in shape))
    return pl.pallas_call(
        _node_body,
        grid=(grid,),
        in_specs=[
            pl.BlockSpec((BR, 128), lambda i: (i, 0)),
            pl.BlockSpec((BR, 128), lambda i: (i, 0)),
            pl.BlockSpec((BR, 128), lambda i: (i, 0)),
            full((1, 1)), full((64, 64)), full((1, 64)), full((64, 128)),
            full((1, 128)), full((64, 128)),
        ],
        out_specs=[pl.BlockSpec((BR, 128), lambda i: (i, 0)),
                   pl.BlockSpec((BR, 128), lambda i: (i, 0))],
        out_shape=[jax.ShapeDtypeStruct((N, 128), jnp.float32),
                   jax.ShapeDtypeStruct((N, 128), jnp.float32)],
    )(h, a0, a1, jnp.full((1, 1), 1.0 + nc['eps'], jnp.float32),
      nc['w1'].T, nc['b1'].reshape(1, 64), _pad128(nc['w2'].T),
      _pad128(nc['b2'].reshape(1, 64)), _pad128(ec_pw.T))


def _node2_body(h_ref, a0_ref, a1_ref, eps_ref, w1_ref, b1_ref, w2_ref, b2_ref,
                hn_ref):
    x = (eps_ref[0, 0] * h_ref[...][:, 0:64] + a0_ref[...][:, 0:64]
         + a1_ref[...][:, 0:64])
    hh = jnp.maximum((jnp.dot(x, w1_ref[...], preferred_element_type=jnp.float32)
                      + b1_ref[...]) * BN, 0.0)
    hn_ref[...] = _leaky((jnp.dot(hh, w2_ref[...],
                                  preferred_element_type=jnp.float32)
                          + b2_ref[...]) * BN)


def _run_node2(h, a0, a1, nc):
    grid = N // BR
    full = lambda shape: pl.BlockSpec(shape, lambda i: tuple(0 for _ in shape))
    return pl.pallas_call(
        _node2_body,
        grid=(grid,),
        in_specs=[
            pl.BlockSpec((BR, 128), lambda i: (i, 0)),
            pl.BlockSpec((BR, 128), lambda i: (i, 0)),
            pl.BlockSpec((BR, 128), lambda i: (i, 0)),
            full((1, 1)), full((64, 64)), full((1, 64)), full((64, 64)),
            full((1, 64)),
        ],
        out_specs=pl.BlockSpec((BR, 64), lambda i: (i, 0)),
        out_shape=jax.ShapeDtypeStruct((N, 64), jnp.float32),
    )(h, a0, a1, jnp.full((1, 1), 1.0 + nc['eps'], jnp.float32),
      nc['w1'].T, nc['b1'].reshape(1, 64), nc['w2'].T, nc['b2'].reshape(1, 64))


def _edge_body(he_ref, g1_ref, g2_ref, eps_ref, pb2_ref, w1_ref, b1_ref,
               w2_ref, b2_ref, npw_ref, npb_ref, ep_ref):
    x = (eps_ref[0, 0] * he_ref[...] + g1_ref[...][:, 0:64]
         + g2_ref[...][:, 0:64] + pb2_ref[...])
    hh = jnp.maximum((jnp.dot(x, w1_ref[...], preferred_element_type=jnp.float32)
                      + b1_ref[...]) * BN, 0.0)
    y = _leaky((jnp.dot(hh, w2_ref[...], preferred_element_type=jnp.float32)
                + b2_ref[...]) * BN)
    ep_ref[...] = jnp.dot(y, npw_ref[...],
                          preferred_element_type=jnp.float32) + npb_ref[...]


def _run_edge(he, g1, g2, ec, nc1):
    grid = E // BR
    full = lambda shape: pl.BlockSpec(shape, lambda i: tuple(0 for _ in shape))
    return pl.pallas_call(
        _edge_body,
        grid=(grid,),
        in_specs=[
            pl.BlockSpec((BR, 64), lambda i: (i, 0)),
            pl.BlockSpec((BR, 128), lambda i: (i, 0)),
            pl.BlockSpec((BR, 128), lambda i: (i, 0)),
            full((1, 1)), full((1, 64)), full((64, 64)), full((1, 64)),
            full((64, 64)), full((1, 64)), full((64, 128)), full((1, 128)),
        ],
        out_specs=pl.BlockSpec((BR, 128), lambda i: (i, 0)),
        out_shape=jax.ShapeDtypeStruct((E, 128), jnp.float32),
    )(he, g1, g2, jnp.full((1, 1), 1.0 + ec['eps'], jnp.float32),
      (2.0 * ec['pb']).reshape(1, 64), ec['w1'].T, ec['b1'].reshape(1, 64),
      ec['w2'].T, ec['b2'].reshape(1, 64), _pad128(nc1['pw'].T),
      _pad128(nc1['pb'].reshape(1, 64)))


GP = 104


def _readout_body(h0_ref, h1_ref, h2_ref, ids_ref, pw0_ref, pw1_ref, pw2_ref,
                  pb_ref, pj1_ref, pj2_ref, pj3_ref, out_ref, p0, p1, p2):
    v0 = h0_ref[...]
    v1 = h1_ref[...]
    v2 = h2_ref[...]
    ids = ids_ref[...]
    neg = jnp.float32(-jnp.inf)

    def body(g, _):
        mask = ids == g
        p0[pl.ds(g, 1), :] = jnp.max(jnp.where(mask, v0, neg), axis=0,
                                     keepdims=True)
        p1[pl.ds(g, 1), :] = jnp.max(jnp.where(mask, v1, neg), axis=0,
                                     keepdims=True)
        p2[pl.ds(g, 1), :] = jnp.max(jnp.where(mask, v2, neg), axis=0,
                                     keepdims=True)
        return 0

    lax.fori_loop(0, G, body, 0)

    score = (jnp.dot(p0[...], pw0_ref[...], preferred_element_type=jnp.float32)
             + jnp.dot(p1[...], pw1_ref[...], preferred_element_type=jnp.float32)
             + jnp.dot(p2[...], pw2_ref[...], preferred_element_type=jnp.float32)
             + pb_ref[...])
    t = jnp.maximum(jnp.dot(score, pj1_ref[...],
                            preferred_element_type=jnp.float32) * BN, 0.0)
    t = jnp.maximum(jnp.dot(t, pj2_ref[...],
                            preferred_element_type=jnp.float32) * BN, 0.0)
    t = jnp.dot(t, pj3_ref[...], preferred_element_type=jnp.float32)
    nrm = jnp.sqrt(jnp.sum(t * t, axis=-1, keepdims=True))
    out = t / jnp.maximum(nrm, 1e-12)
    out_ref[...] = out[0:G, :]


def _run_readout(h0, h1, h2, graph_ids, params):
    pbsum = (params['pred_b'][0] + params['pred_b'][1]
             + params['pred_b'][2]).reshape(1, 128)
    full = lambda shape: pl.BlockSpec(shape, lambda: tuple(0 for _ in shape))
    return pl.pallas_call(
        _readout_body,
        in_specs=[
            full((N, 64)), full((N, 64)), full((N, 64)), full((N, 1)),
            full((64, 128)), full((64, 128)), full((64, 128)), full((1, 128)),
            full((128, 128)), full((128, 128)), full((128, 128)),
        ],
        out_specs=full((G, 128)),
        out_shape=jax.ShapeDtypeStruct((G, 128), jnp.float32),
        scratch_shapes=[
            pltpu.VMEM((GP, 64), jnp.float32),
            pltpu.VMEM((GP, 64), jnp.float32),
            pltpu.VMEM((GP, 64), jnp.float32),
        ],
    )(h0, h1, h2, graph_ids.reshape(N, 1),
      params['pred_w'][0].T, params['pred_w'][1].T, params['pred_w'][2].T,
      pbsum, params['pjw1'].T, params['pjw2'].T, params['pjw3'].T)


def kernel(node_x, edge_x, params, src, dst, graph_ids):
    nfeat = node_x[:, jnp.array([0, 1, 2, 6]), :, :]
    efeat = edge_x[:, :3, :]

    crv, ep0 = _run_curve(efeat, params['cw1'], params['cw2'], params['cw3'],
                          params['cfc'], params['nc'][0]['pw'],
                          params['nc'][0]['pb'])
    srf = _run_surface(nfeat, params['sw1'], params['sw2'], params['sw3'],
                       params['sfc'])

    agg0 = _run_seg(srf, ep0, src, dst)
    hn1, hnp1 = _run_node(srf, agg0[0:N], agg0[N:2 * N],
                          params['nc'][0], params['ec'][0]['pw'])
    g1, g2 = _run_gather2(hnp1, src, dst)
    ep1 = _run_edge(crv, g1, g2, params['ec'][0], params['nc'][1])
    agg1 = _run_seg(hn1, ep1, src, dst)
    hn2 = _run_node2(hn1, agg1[0:N], agg1[N:2 * N], params['nc'][1])

    return _run_readout(srf[:, 0:64], hn1[:, 0:64], hn2, graph_ids, params)


# speedup vs baseline: 1.7572x; 1.7572x over previous
import functools

import jax
import jax.numpy as jnp
from jax import lax
from jax.experimental import pallas as pl
from jax.experimental.pallas import tpu as pltpu
from jax.experimental.pallas import tpu_sc as plsc

N = 10000
E = 160000
G = 100
BN = 1.0 / (1.0 + 1e-5) ** 0.5

NC = 2
NS = 16
NW = NC * NS
CK = 128
NCHUNK = E // CK
ITERS = (NCHUNK + NW - 1) // NW


def _leaky(x):
    return jnp.where(x >= 0, x, 0.01 * x)


def _pad128(w):
    return jnp.pad(w, ((0, 0), (0, 128 - w.shape[1])))


BE = 256


def _curve_body(x_ref, w1_ref, w2_ref, w3_ref, fc_ref, pw_ref, pb_ref,
                crv_ref, ep_ref, s1, s2, s3):
    b = BE

    def fill(s, val, c):
        s[pl.ds(0, b), :] = jnp.zeros((b, c), jnp.float32)
        s[pl.ds(11 * b, b), :] = jnp.zeros((b, c), jnp.float32)
        s[pl.ds(b, 10 * b), :] = val

    y1 = _leaky(jnp.dot(x_ref[...], w1_ref[...],
                        preferred_element_type=jnp.float32) * BN)
    s1[pl.ds(0, b), :] = jnp.zeros((b, 64), jnp.float32)
    s1[pl.ds(11 * b, b), :] = jnp.zeros((b, 64), jnp.float32)
    for l in range(10):
        s1[pl.ds((1 + l) * b, b), :] = y1[:, l * 64:(l + 1) * 64]

    def layer(s_in, s_out, w_ref, cout):
        acc = jnp.dot(s_in[pl.ds(0, 10 * b), :], w_ref[0],
                      preferred_element_type=jnp.float32)
        acc = acc + jnp.dot(s_in[pl.ds(b, 10 * b), :], w_ref[1],
                            preferred_element_type=jnp.float32)
        acc = acc + jnp.dot(s_in[pl.ds(2 * b, 10 * b), :], w_ref[2],
                            preferred_element_type=jnp.float32)
        fill(s_out, _leaky(acc * BN), cout)

    layer(s1, s2, w2_ref, 128)
    layer(s2, s3, w3_ref, 256)

    m = s3[pl.ds(b, b), :]
    for l in range(1, 10):
        m = m + s3[pl.ds((1 + l) * b, b), :]
    m = m * 0.1
    crv = _leaky(jnp.dot(m, fc_ref[...], preferred_element_type=jnp.float32) * BN)
    crv_ref[...] = crv
    ep_ref[...] = jnp.dot(crv, pw_ref[...],
                          preferred_element_type=jnp.float32) + pb_ref[...]


def _curve_band(cw1):
    import numpy as np
    sel = np.zeros((30, 10, 9), np.float32)
    for l in range(10):
        for d in range(3):
            lp = l + d - 1
            if 0 <= lp < 10:
                for c in range(3):
                    sel[c * 10 + lp, l, c * 3 + d] = 1.0
    wflat = jnp.transpose(cw1, (1, 2, 0)).reshape(9, 64)
    return jnp.tensordot(jnp.asarray(sel), wflat, axes=([2], [0])).reshape(30, 640)


def _run_curve(efeat, cw1, cw2, cw3, cfc, pw0, pb0):
    x30 = efeat.reshape(E, 30)
    grid = E // BE
    return pl.pallas_call(
        _curve_body,
        grid=(grid,),
        in_specs=[
            pl.BlockSpec((BE, 30), lambda i: (i, 0)),
            pl.BlockSpec((30, 640), lambda i: (0, 0)),
            pl.BlockSpec((3, 64, 128), lambda i: (0, 0, 0)),
            pl.BlockSpec((3, 128, 256), lambda i: (0, 0, 0)),
            pl.BlockSpec((256, 64), lambda i: (0, 0)),
            pl.BlockSpec((64, 128), lambda i: (0, 0)),
            pl.BlockSpec((1, 128), lambda i: (0, 0)),
        ],
        out_specs=[
            pl.BlockSpec((BE, 64), lambda i: (i, 0)),
            pl.BlockSpec((BE, 128), lambda i: (i, 0)),
        ],
        out_shape=[
            jax.ShapeDtypeStruct((E, 64), jnp.float32),
            jax.ShapeDtypeStruct((E, 128), jnp.float32),
        ],
        scratch_shapes=[
            pltpu.VMEM((12 * BE, 64), jnp.float32),
            pltpu.VMEM((12 * BE, 128), jnp.float32),
            pltpu.VMEM((12 * BE, 256), jnp.float32),
        ],
    )(x30, _curve_band(cw1), jnp.transpose(cw2, (2, 1, 0)),
      jnp.transpose(cw3, (2, 1, 0)), cfc.T, _pad128(pw0.T),
      _pad128(pb0.reshape(1, 64)))


BS = 16


def _surface_body(x_ref, w1_ref, w2_ref, w3_ref, fc_ref, srf_ref, s1, s2, s3):
    b = BS
    rows = 118 * b

    y1 = _leaky(jnp.dot(x_ref[...], w1_ref[...],
                        preferred_element_type=jnp.float32) * BN)
    s1[...] = jnp.zeros((144 * b, 64), jnp.float32)
    for y in range(10):
        for x in range(10):
            q = (y + 1) * 12 + (x + 1)
            v = y * 10 + x
            s1[pl.ds(q * b, b), :] = y1[:, v * 64:(v + 1) * 64]

    def layer(s_in, s_out, w_ref, cout):
        acc = None
        for dy in range(3):
            for dx in range(3):
                sh = ((dy - 1) * 12 + (dx - 1)) * b
                t = jnp.dot(s_in[pl.ds(13 * b + sh, rows), :], w_ref[dy, dx],
                            preferred_element_type=jnp.float32)
                acc = t if acc is None else acc + t
        y = _leaky(acc * BN)
        ridx = lax.broadcasted_iota(jnp.int32, (rows, cout), 0)
        pos = ridx // b + 13
        hh = pos // 12
        ww = pos - hh * 12
        valid = (hh >= 1) & (hh <= 10) & (ww >= 1) & (ww <= 10)
        y = jnp.where(valid, y, 0.0)
        s_out[pl.ds(0, 13 * b), :] = jnp.zeros((13 * b, cout), jnp.float32)
        s_out[pl.ds(131 * b, 13 * b), :] = jnp.zeros((13 * b, cout), jnp.float32)
        s_out[pl.ds(13 * b, rows), :] = y

    layer(s1, s2, w2_ref, 128)
    layer(s2, s3, w3_ref, 256)

    v = s3[...]
    a = v[0:12 * b, :]
    for p in range(1, 12):
        a = a + v[p * 12 * b:(p + 1) * 12 * b, :]
    mrow = a[0:b, :]
    for q in range(1, 12):
        mrow = mrow + a[q * b:(q + 1) * b, :]
    m = mrow * 0.01
    srf_ref[...] = _leaky(jnp.dot(m, fc_ref[...],
                                  preferred_element_type=jnp.float32) * BN)


def _surface_band(sw1):
    import numpy as np
    sel = np.zeros((400, 100, 36), np.float32)
    for y in range(10):
        for x in range(10):
            v = y * 10 + x
            for dy in range(3):
                for dx in range(3):
                    py, px = y + dy - 1, x + dx - 1
                    if 0 <= py < 10 and 0 <= px < 10:
                        for c in range(4):
                            sel[c * 100 + py * 10 + px, v, c * 9 + dy * 3 + dx] = 1.0
    wflat = jnp.transpose(sw1, (1, 2, 3, 0)).reshape(36, 64)
    return jnp.tensordot(jnp.asarray(sel), wflat,
                         axes=([2], [0])).reshape(400, 6400)


def _run_surface(nfeat, sw1, sw2, sw3, sfc):
    x400 = nfeat.reshape(N, 400)
    grid = N // BS
    return pl.pallas_call(
        _surface_body,
        grid=(grid,),
        in_specs=[
            pl.BlockSpec((BS, 400), lambda i: (i, 0)),
            pl.BlockSpec((400, 6400), lambda i: (0, 0)),
            pl.BlockSpec((3, 3, 64, 128), lambda i: (0, 0, 0, 0)),
            pl.BlockSpec((3, 3, 128, 256), lambda i: (0, 0, 0, 0)),
            pl.BlockSpec((256, 128), lambda i: (0, 0)),
        ],
        out_specs=pl.BlockSpec((BS, 128), lambda i: (i, 0)),
        out_shape=jax.ShapeDtypeStruct((N, 128), jnp.float32),
        scratch_shapes=[
            pltpu.VMEM((144 * BS, 64), jnp.float32),
            pltpu.VMEM((144 * BS, 128), jnp.float32),
            pltpu.VMEM((144 * BS, 256), jnp.float32),
        ],
        compiler_params=pltpu.CompilerParams(
            vmem_limit_bytes=100 * 1024 * 1024),
    )(x400, _surface_band(sw1), jnp.transpose(sw2, (2, 3, 1, 0)),
      jnp.transpose(sw3, (2, 3, 1, 0)), _pad128(sfc.T))


def _seg_body(h_hbm, ep_hbm, src_hbm, dst_hbm, zeros_hbm, out_hbm,
              src_v, dst_v, rows_v, ep_v, shared):
    cidx = lax.axis_index("c")
    sid = lax.axis_index("s")
    w = sid * NC + cidx

    @pl.when(sid < 10)
    def _():
        pltpu.sync_copy(zeros_hbm, shared.at[pl.ds(sid * 1000, 1000)])

    plsc.subcore_barrier()

    def body(it, _):
        cid = it * NW + w

        @pl.when(cid < NCHUNK)
        def _():
            base = cid * CK
            pltpu.sync_copy(src_hbm.at[pl.ds(base, CK)], src_v)
            pltpu.sync_copy(dst_hbm.at[pl.ds(base, CK)], dst_v)
            pltpu.sync_copy(h_hbm.at[src_v], rows_v)
            pltpu.sync_copy(ep_hbm.at[pl.ds(base, CK)], ep_v)
            pltpu.sync_copy(rows_v, shared.at[dst_v], add=True)
            pltpu.sync_copy(ep_v, shared.at[dst_v], add=True)
        return 0

    lax.fori_loop(0, ITERS, body, 0)
    plsc.subcore_barrier()

    @pl.when(sid < 10)
    def _():
        pltpu.sync_copy(shared.at[pl.ds(sid * 1000, 1000)],
                        out_hbm.at[pl.ds(cidx * N + sid * 1000, 1000)])


def _run_seg(h, ep, src, dst):
    zeros = jnp.zeros((1000, 128), jnp.float32)
    mesh = plsc.VectorSubcoreMesh(core_axis_name="c", subcore_axis_name="s")
    fn = functools.partial(
        pl.kernel, mesh=mesh,
        out_type=jax.ShapeDtypeStruct((2 * N, 128), jnp.float32),
        scratch_types=[
            pltpu.VMEM((CK,), jnp.int32),
            pltpu.VMEM((CK,), jnp.int32),
            pltpu.VMEM((CK, 128), jnp.float32),
            pltpu.VMEM((CK, 128), jnp.float32),
            pltpu.VMEM_SHARED((N, 128), jnp.float32),
        ],
    )(_seg_body)
    return fn(h, ep, src, dst, zeros)


def _g2_body(t_hbm, src_hbm, dst_hbm, g1_hbm, g2_hbm, src_v, dst_v, r1_v, r2_v):
    cidx = lax.axis_index("c")
    sid = lax.axis_index("s")
    w = sid * NC + cidx

    def body(it, _):
        cid = it * NW + w

        @pl.when(cid < NCHUNK)
        def _():
            base = cid * CK
            pltpu.sync_copy(src_hbm.at[pl.ds(base, CK)], src_v)
            pltpu.sync_copy(dst_hbm.at[pl.ds(base, CK)], dst_v)
            pltpu.sync_copy(t_hbm.at[src_v], r1_v)
            pltpu.sync_copy(t_hbm.at[dst_v], r2_v)
            pltpu.sync_copy(r1_v, g1_hbm.at[pl.ds(base, CK)])
            pltpu.sync_copy(r2_v, g2_hbm.at[pl.ds(base, CK)])
        return 0

    lax.fori_loop(0, ITERS, body, 0)


def _run_gather2(table, src, dst):
    mesh = plsc.VectorSubcoreMesh(core_axis_name="c", subcore_axis_name="s")
    fn = functools.partial(
        pl.kernel, mesh=mesh,
        out_type=[jax.ShapeDtypeStruct((E, 128), jnp.float32),
                  jax.ShapeDtypeStruct((E, 128), jnp.float32)],
        scratch_types=[
            pltpu.VMEM((CK,), jnp.int32),
            pltpu.VMEM((CK,), jnp.int32),
            pltpu.VMEM((CK, 128), jnp.float32),
            pltpu.VMEM((CK, 128), jnp.float32),
        ],
    )(_g2_body)
    return fn(table, src, dst)


BR = 2000


def _node_body(h_ref, a0_ref, a1_ref, eps_ref, w1_ref, b1_ref, w2_ref, b2_ref,
               pw_ref, hn_ref, hnp_ref):
    x = (eps_ref[0, 0] * h_ref[...][:, 0:64] + a0_ref[...][:, 0:64]
         + a1_ref[...][:, 0:64])
    hh = jnp.maximum((jnp.dot(x, w1_ref[...], preferred_element_type=jnp.float32)
                      + b1_ref[...]) * BN, 0.0)
    y = _leaky((jnp.dot(hh, w2_ref[...], preferred_element_type=jnp.float32)
                + b2_ref[...]) * BN)
    hn_ref[...] = y
    hnp_ref[...] = jnp.dot(y[:, 0:64], pw_ref[...],
                           preferred_element_type=jnp.float32)


def _run_node(h, a0, a1, nc, ec_pw):
    grid = N // BR
    full = lambda shape: pl.BlockSpec(shape, lambda i: tuple(0 for _ in shape))
    return pl.pallas_call(
        _node_body,
        grid=(grid,),
        in_specs=[
            pl.BlockSpec((BR, 128), lambda i: (i, 0)),
            pl.BlockSpec((BR, 128), lambda i: (i, 0)),
            pl.BlockSpec((BR, 128), lambda i: (i, 0)),
            full((1, 1)), full((64, 64)), full((1, 64)), full((64, 128)),
            full((1, 128)), full((64, 128)),
        ],
        out_specs=[pl.BlockSpec((BR, 128), lambda i: (i, 0)),
                   pl.BlockSpec((BR, 128), lambda i: (i, 0))],
        out_shape=[jax.ShapeDtypeStruct((N, 128), jnp.float32),
                   jax.ShapeDtypeStruct((N, 128), jnp.float32)],
    )(h, a0, a1, jnp.full((1, 1), 1.0 + nc['eps'], jnp.float32),
      nc['w1'].T, nc['b1'].reshape(1, 64), _pad128(nc['w2'].T),
      _pad128(nc['b2'].reshape(1, 64)), _pad128(ec_pw.T))


def _node2_body(h_ref, a0_ref, a1_ref, eps_ref, w1_ref, b1_ref, w2_ref, b2_ref,
                hn_ref):
    x = (eps_ref[0, 0] * h_ref[...][:, 0:64] + a0_ref[...][:, 0:64]
         + a1_ref[...][:, 0:64])
    hh = jnp.maximum((jnp.dot(x, w1_ref[...], preferred_element_type=jnp.float32)
                      + b1_ref[...]) * BN, 0.0)
    hn_ref[...] = _leaky((jnp.dot(hh, w2_ref[...],
                                  preferred_element_type=jnp.float32)
                          + b2_ref[...]) * BN)


def _run_node2(h, a0, a1, nc):
    grid = N // BR
    full = lambda shape: pl.BlockSpec(shape, lambda i: tuple(0 for _ in shape))
    return pl.pallas_call(
        _node2_body,
        grid=(grid,),
        in_specs=[
            pl.BlockSpec((BR, 128), lambda i: (i, 0)),
            pl.BlockSpec((BR, 128), lambda i: (i, 0)),
            pl.BlockSpec((BR, 128), lambda i: (i, 0)),
            full((1, 1)), full((64, 64)), full((1, 64)), full((64, 64)),
            full((1, 64)),
        ],
        out_specs=pl.BlockSpec((BR, 64), lambda i: (i, 0)),
        out_shape=jax.ShapeDtypeStruct((N, 64), jnp.float32),
    )(h, a0, a1, jnp.full((1, 1), 1.0 + nc['eps'], jnp.float32),
      nc['w1'].T, nc['b1'].reshape(1, 64), nc['w2'].T, nc['b2'].reshape(1, 64))


def _edge_body(he_ref, g1_ref, g2_ref, eps_ref, pb2_ref, w1_ref, b1_ref,
               w2_ref, b2_ref, npw_ref, npb_ref, ep_ref):
    x = (eps_ref[0, 0] * he_ref[...] + g1_ref[...][:, 0:64]
         + g2_ref[...][:, 0:64] + pb2_ref[...])
    hh = jnp.maximum((jnp.dot(x, w1_ref[...], preferred_element_type=jnp.float32)
                      + b1_ref[...]) * BN, 0.0)
    y = _leaky((jnp.dot(hh, w2_ref[...], preferred_element_type=jnp.float32)
                + b2_ref[...]) * BN)
    ep_ref[...] = jnp.dot(y, npw_ref[...],
                          preferred_element_type=jnp.float32) + npb_ref[...]


def _run_edge(he, g1, g2, ec, nc1):
    grid = E // BR
    full = lambda shape: pl.BlockSpec(shape, lambda i: tuple(0 for _ in shape))
    return pl.pallas_call(
        _edge_body,
        grid=(grid,),
        in_specs=[
            pl.BlockSpec((BR, 64), lambda i: (i, 0)),
            pl.BlockSpec((BR, 128), lambda i: (i, 0)),
            pl.BlockSpec((BR, 128), lambda i: (i, 0)),
            full((1, 1)), full((1, 64)), full((64, 64)), full((1, 64)),
            full((64, 64)), full((1, 64)), full((64, 128)), full((1, 128)),
        ],
        out_specs=pl.BlockSpec((BR, 128), lambda i: (i, 0)),
        out_shape=jax.ShapeDtypeStruct((E, 128), jnp.float32),
    )(he, g1, g2, jnp.full((1, 1), 1.0 + ec['eps'], jnp.float32),
      (2.0 * ec['pb']).reshape(1, 64), ec['w1'].T, ec['b1'].reshape(1, 64),
      ec['w2'].T, ec['b2'].reshape(1, 64), _pad128(nc1['pw'].T),
      _pad128(nc1['pb'].reshape(1, 64)))


GP = 104


def _readout_body(h0_ref, h1_ref, h2_ref, ids_ref, pw0_ref, pw1_ref, pw2_ref,
                  pb_ref, pj1_ref, pj2_ref, pj3_ref, out_ref, p0, p1, p2):
    v0 = h0_ref[...]
    v1 = h1_ref[...]
    v2 = h2_ref[...]
    ids = ids_ref[...]
    neg = jnp.float32(-jnp.inf)

    def body(g, _):
        mask = ids == g
        p0[pl.ds(g, 1), :] = jnp.max(jnp.where(mask, v0, neg), axis=0,
                                     keepdims=True)
        p1[pl.ds(g, 1), :] = jnp.max(jnp.where(mask, v1, neg), axis=0,
                                     keepdims=True)
        p2[pl.ds(g, 1), :] = jnp.max(jnp.where(mask, v2, neg), axis=0,
                                     keepdims=True)
        return 0

    lax.fori_loop(0, G, body, 0)

    score = (jnp.dot(p0[...], pw0_ref[...], preferred_element_type=jnp.float32)
             + jnp.dot(p1[...], pw1_ref[...], preferred_element_type=jnp.float32)
             + jnp.dot(p2[...], pw2_ref[...], preferred_element_type=jnp.float32)
             + pb_ref[...])
    t = jnp.maximum(jnp.dot(score, pj1_ref[...],
                            preferred_element_type=jnp.float32) * BN, 0.0)
    t = jnp.maximum(jnp.dot(t, pj2_ref[...],
                            preferred_element_type=jnp.float32) * BN, 0.0)
    t = jnp.dot(t, pj3_ref[...], preferred_element_type=jnp.float32)
    nrm = jnp.sqrt(jnp.sum(t * t, axis=-1, keepdims=True))
    out = t / jnp.maximum(nrm, 1e-12)
    out_ref[...] = out[0:G, :]


def _run_readout(h0, h1, h2, graph_ids, params):
    pbsum = (params['pred_b'][0] + params['pred_b'][1]
             + params['pred_b'][2]).reshape(1, 128)
    full = lambda shape: pl.BlockSpec(shape, lambda: tuple(0 for _ in shape))
    return pl.pallas_call(
        _readout_body,
        in_specs=[
            full((N, 64)), full((N, 64)), full((N, 64)), full((N, 1)),
            full((64, 128)), full((64, 128)), full((64, 128)), full((1, 128)),
            full((128, 128)), full((128, 128)), full((128, 128)),
        ],
        out_specs=full((G, 128)),
        out_shape=jax.ShapeDtypeStruct((G, 128), jnp.float32),
        scratch_shapes=[
            pltpu.VMEM((GP, 64), jnp.float32),
            pltpu.VMEM((GP, 64), jnp.float32),
            pltpu.VMEM((GP, 64), jnp.float32),
        ],
    )(h0, h1, h2, graph_ids.reshape(N, 1),
      params['pred_w'][0].T, params['pred_w'][1].T, params['pred_w'][2].T,
      pbsum, params['pjw1'].T, params['pjw2'].T, params['pjw3'].T)


def kernel(node_x, edge_x, params, src, dst, graph_ids):
    nfeat = node_x[:, jnp.array([0, 1, 2, 6]), :, :]
    efeat = edge_x[:, :3, :]

    crv, ep0 = _run_curve(efeat, params['cw1'], params['cw2'], params['cw3'],
                          params['cfc'], params['nc'][0]['pw'],
                          params['nc'][0]['pb'])
    srf = _run_surface(nfeat, params['sw1'], params['sw2'], params['sw3'],
                       params['sfc'])

    agg0 = _run_seg(srf, ep0, src, dst)
    hn1, hnp1 = _run_node(srf, agg0[0:N], agg0[N:2 * N],
                          params['nc'][0], params['ec'][0]['pw'])
    g1, g2 = _run_gather2(hnp1, src, dst)
    ep1 = _run_edge(crv, g1, g2, params['ec'][0], params['nc'][1])
    agg1 = _run_seg(hn1, ep1, src, dst)
    hn2 = _run_node2(hn1, agg1[0:N], agg1[N:2 * N], params['nc'][1])

    return _run_readout(srf[:, 0:64], hn1[:, 0:64], hn2, graph_ids, params)
